# Initial kernel scaffold; baseline (speedup 1.0000x reference)
import functools

import jax
import jax.numpy as jnp
from jax import lax
from jax.experimental import pallas as pl
from jax.experimental.pallas import tpu as pltpu
from jax.experimental.pallas import tpu_sc as plsc

DIM = 96
D2 = DIM * DIM
LINSZ = DIM ** 3
CCH = 16
NBKT = 216
BSH = 12
BSZ = 1 << BSH
HP = 224
NG = 500000
NC = 200000
WIN = 512
L = 16


def _vg(a, ix):
    return jnp.take(a, ix, axis=0, mode=lax.GatherScatterMode.PROMISE_IN_BOUNDS)


def _consts():
    i16 = lax.iota(jnp.int32, L)
    c = lambda v: jnp.clip(v, 0, L - 1)
    return dict(
        i16=i16,
        m6=i16 < 6, m11=i16 < 11, m5=i16 < 5, m10=i16 < 10,
        IAX=c(3 * i16), IBX=c(3 * i16 - 16), ICX=c(3 * i16 - 32),
        IAY=c(3 * i16 + 1), IBY=c(3 * i16 - 15), ICY=c(3 * i16 - 31),
        IAZ=c(3 * i16 + 2), IBZ=c(3 * i16 - 14), ICZ=c(3 * i16 - 30),
    )


def _deinterleave(cbuf, g, k):
    a = cbuf[pl.ds(g * 48, L)]
    b = cbuf[pl.ds(g * 48 + L, L)]
    c3 = cbuf[pl.ds(g * 48 + 2 * L, L)]
    xx = jnp.where(k["m6"], _vg(a, k["IAX"]),
                   jnp.where(k["m11"], _vg(b, k["IBX"]), _vg(c3, k["ICX"])))
    yy = jnp.where(k["m5"], _vg(a, k["IAY"]),
                   jnp.where(k["m11"], _vg(b, k["IBY"]), _vg(c3, k["ICY"])))
    zz = jnp.where(k["m5"], _vg(a, k["IAZ"]),
                   jnp.where(k["m10"], _vg(b, k["IBZ"]), _vg(c3, k["ICZ"])))
    return xx * D2 + yy * DIM + zz


def _starts_tots(h_ref, histall, st_buf, tt_buf, nw):
    pltpu.sync_copy(h_ref, histall)
    carry = jnp.int32(0)
    for kb in range(HP // L):
        sl = pl.ds(kb * L, L)
        tot = jnp.zeros((L,), jnp.int32)
        for t in range(nw):
            tot = tot + histall[t, sl]
        ptot = jnp.bitwise_and(tot + 7, -8)
        cum = plsc.cumsum(ptot)
        st_buf[sl] = cum - ptot + carry
        tt_buf[sl] = tot
        carry = carry + jnp.sum(ptot)


def _my_bases(wid, histall, st_buf, brun, nw):
    for kb in range(HP // L):
        sl = pl.ds(kb * L, L)
        myb = st_buf[sl]
        for t in range(nw):
            myb = myb + jnp.where(jnp.int32(t) < wid, histall[t, sl],
                                  jnp.zeros((L,), jnp.int32))
        brun[sl] = myb


def kernel(global_coords, global_values, current_coords, current_values,
           Wz, Uz, Wr, Ur, Wn, Un, bz, br, bn):
    info = plsc.get_sparse_core_info()
    nw = info.num_cores * info.num_subcores
    mesh = plsc.VectorSubcoreMesh(core_axis_name="c", subcore_axis_name="s")

    def roundup(a, b):
        return (a + b - 1) // b * b

    CH_G = roundup((NG + nw - 1) // nw, WIN)
    CH_C = roundup((NC + nw - 1) // nw, WIN)
    NGP = CH_G * nw
    NCP = CH_C * nw
    RG_SZ = NG + NBKT * 8 + WIN
    RC_SZ = NC + NBKT * 8 + WIN
    NBPT = (NBKT + nw - 1) // nw

    @functools.partial(
        pl.kernel,
        out_type=(jax.ShapeDtypeStruct((NGP,), jnp.int32),
                  jax.ShapeDtypeStruct((NCP,), jnp.int32),
                  jax.ShapeDtypeStruct((nw, HP), jnp.int32),
                  jax.ShapeDtypeStruct((nw, HP), jnp.int32)),
        mesh=mesh,
        scratch_types=[pltpu.VMEM((3 * WIN,), jnp.int32),
                       pltpu.VMEM((WIN,), jnp.int32),
                       pltpu.VMEM((HP,), jnp.int32)],
    )
    def k1(cg_ref, cc_ref, lg_ref, lc_ref, hg_ref, hc_ref, cbuf, lbuf, hbuf):
        wid = lax.axis_index("s") * info.num_cores + lax.axis_index("c")
        k = _consts()
        i16 = k["i16"]

        def phase(c_ref, lin_ref, hrow_ref, n, chk):
            def zb(q, c):
                hbuf[pl.ds(q * L, L)] = jnp.zeros((L,), jnp.int32)
                return c
            lax.fori_loop(0, HP // L, zb, 0)
            base = wid * chk

            def wbody(w, c):
                wb = base + w * WIN
                pltpu.sync_copy(c_ref.at[pl.ds(wb * 3, WIN * 3)], cbuf)
                for g in range(WIN // L):
                    linv = _deinterleave(cbuf, g, k)
                    lbuf[pl.ds(g * L, L)] = linv
                    valid = (wb + g * L + i16) < n
                    key = jnp.minimum(linv >> BSH, NBKT - 1)
                    gth = plsc.load_gather(hbuf, [key], mask=valid)
                    cnt, lastm = plsc.scan_count(key, mask=valid)
                    plsc.store_scatter(hbuf, [key], gth + cnt, mask=lastm)
                pltpu.sync_copy(lbuf, lin_ref.at[pl.ds(wb, WIN)])
                return c
            lax.fori_loop(0, chk // WIN, wbody, 0)
            pltpu.sync_copy(hbuf, hrow_ref)

        phase(cg_ref, lg_ref, hg_ref.at[wid], NG, CH_G)
        phase(cc_ref, lc_ref, hc_ref.at[wid], NC, CH_C)

    @functools.partial(
        pl.kernel,
        out_type=(jax.ShapeDtypeStruct((RG_SZ,), jnp.int32),
                  jax.ShapeDtypeStruct((RC_SZ,), jnp.int32)),
        mesh=mesh,
        scratch_types=[pltpu.VMEM((nw, HP), jnp.int32),
                       pltpu.VMEM((HP,), jnp.int32),
                       pltpu.VMEM((HP,), jnp.int32),
                       pltpu.VMEM((HP,), jnp.int32),
                       pltpu.VMEM((WIN,), jnp.int32),
                       pltpu.VMEM((WIN // 128, 128), jnp.int32),
                       pltpu.VMEM((WIN,), jnp.int32),
                       pltpu.SemaphoreType.DMA],
    )
    def k2(lg_ref, lc_ref, hg_ref, hc_ref, rg_ref, rc_ref,
           histall, st_buf, tt_buf, brun, lwin, posb, valb, sem):
        wid = lax.axis_index("s") * info.num_cores + lax.axis_index("c")
        i16 = lax.iota(jnp.int32, L)

        def phase(h_ref, lin_ref, routed_ref, n, chk):
            _starts_tots(h_ref, histall, st_buf, tt_buf, nw)
            _my_bases(wid, histall, st_buf, brun, nw)

            def wbody(w, c):
                wb = wid * chk + w * WIN
                pltpu.sync_copy(lin_ref.at[pl.ds(wb, WIN)], lwin)
                for g in range(WIN // L):
                    linv = lwin[pl.ds(g * L, L)]
                    j = wb + g * L + i16
                    valid = j < n
                    key = jnp.minimum(linv >> BSH, NBKT - 1)
                    off = linv & (BSZ - 1)
                    gb = plsc.load_gather(brun, [key], mask=valid)
                    cnt, lastm = plsc.scan_count(key, mask=valid)
                    plsc.store_scatter(brun, [key], gb + cnt, mask=lastm)
                    posb[g // 8, pl.ds((g % 8) * L, L)] = (
                        jnp.where(valid, gb + cnt - 1, -1))
                    valb[pl.ds(g * L, L)] = j * BSZ + off
                descs = []
                for c8 in range(WIN // 128):
                    descs.append(pltpu.async_copy(
                        valb.at[pl.ds(c8 * 128, 128)],
                        routed_ref.at[plsc.Indices(posb.at[c8],
                                                   ignored_value=-1)],
                        sem))
                for d in descs:
                    d.wait()
                return c
            lax.fori_loop(0, chk // WIN, wbody, 0)

        phase(hg_ref, lg_ref, rg_ref, NG, CH_G)
        phase(hc_ref, lc_ref, rc_ref, NC, CH_C)

    @functools.partial(
        pl.kernel,
        out_type=(jax.ShapeDtypeStruct((RC_SZ, CCH), jnp.float32),
                  jax.ShapeDtypeStruct((RC_SZ, CCH), jnp.float32),
                  jax.ShapeDtypeStruct((RC_SZ,), jnp.int32)),
        mesh=mesh,
        scratch_types=[pltpu.VMEM((nw, HP), jnp.int32),
                       pltpu.VMEM((HP,), jnp.int32),
                       pltpu.VMEM((HP,), jnp.int32),
                       pltpu.VMEM((HP,), jnp.int32),
                       pltpu.VMEM((HP,), jnp.int32),
                       pltpu.VMEM((BSZ,), jnp.int32),
                       pltpu.VMEM((BSZ,), jnp.int32),
                       pltpu.VMEM((WIN,), jnp.int32),
                       pltpu.VMEM((WIN,), jnp.int32),
                       pltpu.VMEM((WIN,), jnp.int32),
                       pltpu.VMEM((WIN,), jnp.int32),
                       pltpu.VMEM((WIN, CCH), jnp.float32),
                       pltpu.VMEM((WIN, CCH), jnp.float32),
                       pltpu.SemaphoreType.DMA],
    )
    def k3(rg_ref, rc_ref, hg_ref, hc_ref, gv_ref, cv_ref,
           h_ref, x_ref, hf_ref,
           histall, stg_b, ttg_b, stc_b, ttc_b, Wg, Wc, rwin,
           hfb, hjb, xib, hrow, xrow, sem):
        wid = lax.axis_index("s") * info.num_cores + lax.axis_index("c")
        i16 = lax.iota(jnp.int32, L)
        _starts_tots(hg_ref, histall, stg_b, ttg_b, nw)
        _starts_tots(hc_ref, histall, stc_b, ttc_b, nw)

        def bucket(it, cc):
            kb = wid + it * nw
            stg = stg_b[kb]
            ttg = ttg_b[kb]
            stc = stc_b[kb]
            ttc = ttc_b[kb]

            def ib(q, c):
                for j in range(L):
                    sl = pl.ds((q * L + j) * L, L)
                    Wg[sl] = jnp.full((L,), -1, jnp.int32)
                    Wc[sl] = jnp.full((L,), -1, jnp.int32)
                return c
            lax.fori_loop(0, BSZ // (L * L), ib, 0)

            def winner(routed_ref, st, tt, wref):
                def wb_(w, c):
                    wb = st + w * WIN
                    pltpu.sync_copy(routed_ref.at[pl.ds(wb, WIN)], rwin)
                    rel = wb - st
                    for g in range(WIN // L):
                        rv = rwin[pl.ds(g * L, L)]
                        e = rel + g * L + i16
                        valid = e < tt
                        off = rv & (BSZ - 1)
                        idxv = rv >> BSH
                        cnt, lastm = plsc.scan_count(off, mask=valid)
                        plsc.store_scatter(wref, [off], idxv, mask=lastm)
                    return c
                lax.fori_loop(0, (tt + WIN - 1) // WIN, wb_, 0)

            winner(rg_ref, stg, ttg, Wg)
            winner(rc_ref, stc, ttc, Wc)

            def ebody(w, c):
                wb = stc + w * WIN
                pltpu.sync_copy(rc_ref.at[pl.ds(wb, WIN)], rwin)
                for g in range(WIN // L):
                    sl = pl.ds(g * L, L)
                    rv = rwin[sl]
                    off = rv & (BSZ - 1)
                    hj = plsc.load_gather(Wg, [off])
                    xi = plsc.load_gather(Wc, [off])
                    hfb[sl] = hj
                    hjb[sl] = jnp.clip(hj, 0, NG - 1)
                    xib[sl] = jnp.clip(xi, 0, NC - 1)
                d1 = pltpu.async_copy(gv_ref.at[hjb], hrow, sem)
                d2 = pltpu.async_copy(cv_ref.at[xib], xrow, sem)
                d1.wait()
                d2.wait()
                pltpu.sync_copy(hrow, h_ref.at[pl.ds(wb, WIN)])
                pltpu.sync_copy(xrow, x_ref.at[pl.ds(wb, WIN)])
                pltpu.sync_copy(hfb, hf_ref.at[pl.ds(wb, WIN)])
                return c
            lax.fori_loop(0, (ttc + WIN - 1) // WIN, ebody, 0)
            return cc
        lax.fori_loop(0, NBPT, bucket, 0)

    R5 = RC_SZ // 8
    B5 = 1024

    def gru_body(h_ref, x_ref, hj_ref, e_ref, wz_r, uz_r, wr_r, ur_r,
                 wn_r, un_r, bz_r, br_r, bn_r, o_ref):
        m = (hj_ref[...] >= 0).astype(jnp.float32)

        def mm(a, w):
            return lax.dot_general(a, w, (((1,), (0,)), ((), ())),
                                   preferred_element_type=jnp.float32,
                                   precision=lax.Precision.HIGHEST)
        m128 = mm(m, e_ref[...])
        h = h_ref[...] * m128
        x = x_ref[...]
        z = jax.nn.sigmoid(mm(x, wz_r[...]) + mm(h, uz_r[...]) + bz_r[...])
        r = jax.nn.sigmoid(mm(x, wr_r[...]) + mm(h, ur_r[...]) + br_r[...])
        nn = jnp.tanh(mm(x, wn_r[...]) + mm(r * h, un_r[...]) + bn_r[...])
        o_ref[...] = (1.0 - z) * h + z * nn

    full = lambda s: pl.BlockSpec(s, lambda i: (0, 0))
    k4 = pl.pallas_call(
        gru_body,
        grid=(pl.cdiv(R5, B5),),
        in_specs=[pl.BlockSpec((B5, 128), lambda i: (i, 0)),
                  pl.BlockSpec((B5, 128), lambda i: (i, 0)),
                  pl.BlockSpec((B5, 8), lambda i: (i, 0)),
                  full((8, 128)),
                  full((128, 128)), full((128, 128)), full((128, 128)),
                  full((128, 128)), full((128, 128)), full((128, 128)),
                  full((1, 128)), full((1, 128)), full((1, 128))],
        out_specs=pl.BlockSpec((B5, 128), lambda i: (i, 0)),
        out_shape=jax.ShapeDtypeStruct((R5, 128), jnp.float32),
    )

    @functools.partial(
        pl.kernel,
        out_type=jax.ShapeDtypeStruct((LINSZ, CCH), jnp.float32),
        mesh=mesh,
        scratch_types=[pltpu.VMEM((nw, HP), jnp.int32),
                       pltpu.VMEM((HP,), jnp.int32),
                       pltpu.VMEM((HP,), jnp.int32),
                       pltpu.VMEM((HP,), jnp.int32),
                       pltpu.VMEM((HP,), jnp.int32),
                       pltpu.VMEM((BSZ,), jnp.int32),
                       pltpu.VMEM((BSZ,), jnp.int32),
                       pltpu.VMEM((WIN,), jnp.int32),
                       pltpu.VMEM((BSZ,), jnp.int32),
                       pltpu.VMEM((BSZ,), jnp.int32),
                       pltpu.VMEM((BSZ // WIN, WIN), jnp.int32),
                       pltpu.VMEM((BSZ // WIN, WIN), jnp.int32),
                       pltpu.VMEM((BSZ // WIN, WIN), jnp.int32),
                       pltpu.VMEM((BSZ, CCH), jnp.float32),
                       pltpu.VMEM((WIN, CCH), jnp.float32),
                       pltpu.SemaphoreType.DMA],
    )
    def k5(rg_ref, rc_ref, hg_ref, hc_ref, gv_ref, hn_ref, vol_ref,
           histall, stg_b, ttg_b, stc_b, ttc_b, Wg, Wc, rwin,
           gg, hgb, gc2d, hc2d, zi2d, grows, zrow, sem):
        wid = lax.axis_index("s") * info.num_cores + lax.axis_index("c")
        i16 = lax.iota(jnp.int32, L)
        _starts_tots(hg_ref, histall, stg_b, ttg_b, nw)
        _starts_tots(hc_ref, histall, stc_b, ttc_b, nw)

        def zb(q, c):
            for j in range(L):
                zrow[q * L + j, :] = jnp.zeros((L,), jnp.float32)
            return c
        lax.fori_loop(0, WIN // L, zb, 0)

        def bucket(it, cc):
            kb = wid + it * nw
            cell0 = kb * BSZ
            stg = stg_b[kb]
            ttg = ttg_b[kb]
            stc = stc_b[kb]
            ttc = ttc_b[kb]

            def ib(q, c):
                for j in range(L):
                    sl = pl.ds((q * L + j) * L, L)
                    Wg[sl] = jnp.full((L,), -1, jnp.int32)
                    Wc[sl] = jnp.full((L,), -1, jnp.int32)
                return c
            lax.fori_loop(0, BSZ // (L * L), ib, 0)

            def wgb(w, c):
                wb = stg + w * WIN
                pltpu.sync_copy(rg_ref.at[pl.ds(wb, WIN)], rwin)
                rel = wb - stg
                for g in range(WIN // L):
                    rv = rwin[pl.ds(g * L, L)]
                    e = rel + g * L + i16
                    valid = e < ttg
                    off = rv & (BSZ - 1)
                    idxv = rv >> BSH
                    cnt, lastm = plsc.scan_count(off, mask=valid)
                    plsc.store_scatter(Wg, [off], idxv, mask=lastm)
                return c
            lax.fori_loop(0, (ttg + WIN - 1) // WIN, wgb, 0)

            def wcb(w, c):
                wb = stc + w * WIN
                pltpu.sync_copy(rc_ref.at[pl.ds(wb, WIN)], rwin)
                rel = wb - stc
                for g in range(WIN // L):
                    rv = rwin[pl.ds(g * L, L)]
                    e = rel + g * L + i16
                    valid = e < ttc
                    off = rv & (BSZ - 1)
                    pos = wb + g * L + i16
                    cnt, lastm = plsc.scan_count(off, mask=valid)
                    plsc.store_scatter(Wc, [off], pos, mask=lastm)
                return c
            lax.fori_loop(0, (ttc + WIN - 1) // WIN, wcb, 0)

            def bp(q, c):
                for j8 in range(8):
                    v = q * 8 + j8
                    sl = pl.ds(v * L, L)
                    wg = Wg[sl]
                    wc = Wc[sl]
                    cella = cell0 + v * L + i16
                    hasg = wg >= 0
                    hasc = wc >= 0
                    inb = cella < LINSZ
                    gsel = hasg & (~hasc) & inb
                    hsel = hasc & inb
                    zsel = (~hasg) & (~hasc) & inb
                    gg[sl] = jnp.where(gsel, wg, -1)
                    hgb[sl] = jnp.where(hsel, wc, -1)
                    row = v // (WIN // L)
                    col = pl.ds((v % (WIN // L)) * L, L)
                    gc2d[row, col] = jnp.where(gsel, cella, -1)
                    hc2d[row, col] = jnp.where(hsel, cella, -1)
                    zi2d[row, col] = jnp.where(zsel, cella, -1)
                return c
            lax.fori_loop(0, BSZ // (L * 8), bp, 0)

            pltpu.async_copy(
                gv_ref.at[plsc.Indices(gg, ignored_value=-1)], grows,
                sem).wait()
            descs = []
            for c8 in range(BSZ // WIN):
                descs.append(pltpu.async_copy(
                    grows.at[pl.ds(c8 * WIN, WIN)],
                    vol_ref.at[plsc.Indices(gc2d.at[c8], ignored_value=-1)],
                    sem))
            for d in descs:
                d.wait()
            pltpu.async_copy(
                hn_ref.at[plsc.Indices(hgb, ignored_value=-1)], grows,
                sem).wait()
            descs = []
            for c8 in range(BSZ // WIN):
                descs.append(pltpu.async_copy(
                    grows.at[pl.ds(c8 * WIN, WIN)],
                    vol_ref.at[plsc.Indices(hc2d.at[c8], ignored_value=-1)],
                    sem))
                descs.append(pltpu.async_copy(
                    zrow,
                    vol_ref.at[plsc.Indices(zi2d.at[c8], ignored_value=-1)],
                    sem))
            for d in descs:
                d.wait()
            return cc
        lax.fori_loop(0, NBPT, bucket, 0)

    cg = global_coords.astype(jnp.int32).reshape(-1)
    cc = current_coords.astype(jnp.int32).reshape(-1)
    cg = jnp.pad(cg, (0, NGP * 3 - NG * 3))
    cc = jnp.pad(cc, (0, NCP * 3 - NC * 3))

    lin_g, lin_c, hist_g, hist_c = k1(cg, cc)
    routed_g, routed_c = k2(lin_g, lin_c, hist_g, hist_c)
    h, x, hjf = k3(routed_g, routed_c, hist_g, hist_c,
                   global_values, current_values)

    eye8 = jnp.eye(8, dtype=jnp.float32)
    kron = lambda w: jnp.kron(eye8, w)
    emask = jnp.kron(eye8, jnp.ones((1, CCH), jnp.float32))
    hn2 = k4(h.reshape(R5, 128), x.reshape(R5, 128), hjf.reshape(R5, 8),
             emask, kron(Wz), kron(Uz), kron(Wr), kron(Ur), kron(Wn),
             kron(Un), jnp.tile(bz, 8)[None], jnp.tile(br, 8)[None],
             jnp.tile(bn, 8)[None])
    hnew = hn2.reshape(RC_SZ, CCH)

    vol = k5(routed_g, routed_c, hist_g, hist_c, global_values, hnew)
    return vol.reshape(DIM, DIM, DIM, CCH)


# speedup vs baseline: 3.6562x; 3.6562x over previous
import functools

import jax
import jax.numpy as jnp
from jax import lax
from jax.experimental import pallas as pl
from jax.experimental.pallas import tpu as pltpu
from jax.experimental.pallas import tpu_sc as plsc

DIM = 96
D2 = DIM * DIM
LINSZ = DIM ** 3
CCH = 16
NBKT = 216
BSH = 12
BSZ = 1 << BSH
HP = 240
NG = 500000
NC = 200000
WIN = 512
L = 16


def _sload(ref, idx):
    return ref[pl.ds(idx, L)][0]


def _deinterleave(cbuf, g, i3):
    xx = plsc.load_gather(cbuf, [i3 + (g * 48)])
    yy = plsc.load_gather(cbuf, [i3 + (g * 48 + 1)])
    zz = plsc.load_gather(cbuf, [i3 + (g * 48 + 2)])
    return xx * D2 + yy * DIM + zz


def _starts_tots(h_ref, histall, st_buf, tt_buf, nw):
    pltpu.sync_copy(h_ref, histall)
    carry = jnp.int32(0)
    for kb in range(HP // L):
        sl = pl.ds(kb * L, L)
        tot = jnp.zeros((L,), jnp.int32)
        for t in range(nw):
            tot = tot + histall[t, sl]
        ptot = jnp.bitwise_and(tot + (WIN - 1), -WIN)
        cum = plsc.cumsum(ptot)
        st_buf[sl] = cum - ptot + carry
        tt_buf[sl] = tot
        carry = carry + jnp.sum(ptot)


def _my_bases(wid, histall, st_buf, brun, nw):
    for kb in range(HP // L):
        sl = pl.ds(kb * L, L)
        myb = st_buf[sl]
        for t in range(nw):
            myb = myb + jnp.where(jnp.int32(t) < wid, histall[t, sl],
                                  jnp.zeros((L,), jnp.int32))
        brun[sl] = myb


def kernel(global_coords, global_values, current_coords, current_values,
           Wz, Uz, Wr, Ur, Wn, Un, bz, br, bn):
    info = plsc.get_sparse_core_info()
    nw = info.num_cores * info.num_subcores
    mesh = plsc.VectorSubcoreMesh(core_axis_name="c", subcore_axis_name="s")

    def roundup(a, b):
        return (a + b - 1) // b * b

    CH_G = roundup((NG + nw - 1) // nw, WIN)
    CH_C = roundup((NC + nw - 1) // nw, WIN)
    NGP = CH_G * nw
    NCP = CH_C * nw
    RG_SZ = NG + NBKT * WIN + WIN
    RC_SZ = NC + NBKT * WIN + WIN
    NBPT = (NBKT + nw - 1) // nw

    @functools.partial(
        pl.kernel,
        out_type=(jax.ShapeDtypeStruct((NGP,), jnp.int32),
                  jax.ShapeDtypeStruct((NCP,), jnp.int32),
                  jax.ShapeDtypeStruct((nw, HP), jnp.int32),
                  jax.ShapeDtypeStruct((nw, HP), jnp.int32)),
        mesh=mesh,
        compiler_params=pltpu.CompilerParams(needs_layout_passes=False, use_tc_tiling_on_sc=False),
        scratch_types=[pltpu.VMEM((3 * WIN,), jnp.int32),
                       pltpu.VMEM((WIN,), jnp.int32),
                       pltpu.VMEM((HP,), jnp.int32)],
    )
    def k1(cg_ref, cc_ref, lg_ref, lc_ref, hg_ref, hc_ref, cbuf, lbuf, hbuf):
        wid = lax.axis_index("s") * info.num_cores + lax.axis_index("c")
        i16 = lax.iota(jnp.int32, L)
        i3 = 3 * i16

        def phase(c_ref, lin_ref, hrow_ref, n, chk):
            def zb(q, c):
                hbuf[pl.ds(q * L, L)] = jnp.zeros((L,), jnp.int32)
                return c
            lax.fori_loop(0, HP // L, zb, 0)
            base = wid * chk

            def wbody(w, c):
                wb = pl.multiple_of(base + w * WIN, WIN)
                pltpu.sync_copy(c_ref.at[pl.ds(wb * 3, WIN * 3)], cbuf)
                for g in range(WIN // L):
                    linv = _deinterleave(cbuf, g, i3)
                    lbuf[pl.ds(g * L, L)] = linv
                    valid = (wb + g * L + i16) < n
                    key = jnp.minimum(linv >> BSH, NBKT - 1)
                    gth = plsc.load_gather(hbuf, [key], mask=valid)
                    cnt, lastm = plsc.scan_count(key, mask=valid)
                    plsc.store_scatter(hbuf, [key], gth + cnt, mask=lastm & valid)
                pltpu.sync_copy(lbuf, lin_ref.at[pl.ds(wb, WIN)])
                return c
            lax.fori_loop(0, chk // WIN, wbody, 0)
            pltpu.sync_copy(hbuf, hrow_ref)

        phase(cg_ref, lg_ref, hg_ref.at[wid], NG, CH_G)
        phase(cc_ref, lc_ref, hc_ref.at[wid], NC, CH_C)

    @functools.partial(
        pl.kernel,
        out_type=(jax.ShapeDtypeStruct((RG_SZ,), jnp.int32),
                  jax.ShapeDtypeStruct((RC_SZ,), jnp.int32)),
        mesh=mesh,
        compiler_params=pltpu.CompilerParams(needs_layout_passes=False, use_tc_tiling_on_sc=False),
        scratch_types=[pltpu.VMEM((nw, HP), jnp.int32),
                       pltpu.VMEM((HP,), jnp.int32),
                       pltpu.VMEM((HP,), jnp.int32),
                       pltpu.VMEM((HP,), jnp.int32),
                       pltpu.VMEM((WIN,), jnp.int32),
                       pltpu.VMEM((WIN // 128, 128), jnp.int32),
                       pltpu.VMEM((WIN,), jnp.int32),
                       pltpu.SemaphoreType.DMA],
    )
    def k2(lg_ref, lc_ref, hg_ref, hc_ref, rg_ref, rc_ref,
           histall, st_buf, tt_buf, brun, lwin, posb, valb, sem):
        wid = lax.axis_index("s") * info.num_cores + lax.axis_index("c")
        i16 = lax.iota(jnp.int32, L)

        def phase(h_ref, lin_ref, routed_ref, n, chk):
            _starts_tots(h_ref, histall, st_buf, tt_buf, nw)
            _my_bases(wid, histall, st_buf, brun, nw)

            def wbody(w, c):
                wb = pl.multiple_of(wid * chk + w * WIN, WIN)
                pltpu.sync_copy(lin_ref.at[pl.ds(wb, WIN)], lwin)
                for g in range(WIN // L):
                    linv = lwin[pl.ds(g * L, L)]
                    j = wb + g * L + i16
                    valid = j < n
                    key = jnp.minimum(linv >> BSH, NBKT - 1)
                    off = linv & (BSZ - 1)
                    gb = plsc.load_gather(brun, [key], mask=valid)
                    cnt, lastm = plsc.scan_count(key, mask=valid)
                    plsc.store_scatter(brun, [key], gb + cnt, mask=lastm & valid)
                    posb[g // 8, pl.ds((g % 8) * L, L)] = (
                        jnp.where(valid, gb + cnt - 1, -1))
                    valb[pl.ds(g * L, L)] = j * BSZ + off
                descs = []
                for c8 in range(WIN // 128):
                    descs.append(pltpu.async_copy(
                        valb.at[pl.ds(c8 * 128, 128)],
                        routed_ref.at[plsc.Indices(posb.at[c8],
                                                   ignored_value=-1)],
                        sem))
                for d in descs:
                    d.wait()
                return c
            lax.fori_loop(0, chk // WIN, wbody, 0)

        phase(hg_ref, lg_ref, rg_ref, NG, CH_G)
        phase(hc_ref, lc_ref, rc_ref, NC, CH_C)

    @functools.partial(
        pl.kernel,
        out_type=(jax.ShapeDtypeStruct((RC_SZ, CCH), jnp.float32),
                  jax.ShapeDtypeStruct((RC_SZ, CCH), jnp.float32),
                  jax.ShapeDtypeStruct((RC_SZ,), jnp.int32)),
        mesh=mesh,
        compiler_params=pltpu.CompilerParams(needs_layout_passes=False, use_tc_tiling_on_sc=False),
        scratch_types=[pltpu.VMEM((nw, HP), jnp.int32),
                       pltpu.VMEM((HP,), jnp.int32),
                       pltpu.VMEM((HP,), jnp.int32),
                       pltpu.VMEM((HP,), jnp.int32),
                       pltpu.VMEM((HP,), jnp.int32),
                       pltpu.VMEM((BSZ,), jnp.int32),
                       pltpu.VMEM((BSZ,), jnp.int32),
                       pltpu.VMEM((WIN,), jnp.int32),
                       pltpu.VMEM((WIN,), jnp.int32),
                       pltpu.VMEM((WIN,), jnp.int32),
                       pltpu.VMEM((WIN,), jnp.int32),
                       pltpu.VMEM((WIN, CCH), jnp.float32),
                       pltpu.VMEM((WIN, CCH), jnp.float32),
                       pltpu.SemaphoreType.DMA],
    )
    def k3(rg_ref, rc_ref, hg_ref, hc_ref, gv_ref, cv_ref,
           h_ref, x_ref, hf_ref,
           histall, stg_b, ttg_b, stc_b, ttc_b, Wg, Wc, rwin,
           hfb, hjb, xib, hrow, xrow, sem):
        wid = lax.axis_index("s") * info.num_cores + lax.axis_index("c")
        i16 = lax.iota(jnp.int32, L)
        _starts_tots(hg_ref, histall, stg_b, ttg_b, nw)
        _starts_tots(hc_ref, histall, stc_b, ttc_b, nw)

        def bucket(it, cc):
            kb = wid + it * nw
            stg = _sload(stg_b, kb)
            ttg = _sload(ttg_b, kb)
            stc = _sload(stc_b, kb)
            ttc = _sload(ttc_b, kb)

            def ib(q, c):
                for j in range(L):
                    sl = pl.ds((q * L + j) * L, L)
                    Wg[sl] = jnp.full((L,), -1, jnp.int32)
                    Wc[sl] = jnp.full((L,), -1, jnp.int32)
                return c
            lax.fori_loop(0, BSZ // (L * L), ib, 0)

            def winner(routed_ref, st, tt, wref):
                def wb_(w, c):
                    wb = pl.multiple_of(st + w * WIN, 8)
                    pltpu.sync_copy(routed_ref.at[pl.ds(wb, WIN)], rwin)
                    rel = wb - st
                    for g in range(WIN // L):
                        rv = rwin[pl.ds(g * L, L)]
                        e = rel + g * L + i16
                        valid = e < tt
                        off = rv & (BSZ - 1)
                        idxv = rv >> BSH
                        cnt, lastm = plsc.scan_count(off, mask=valid)
                        plsc.store_scatter(wref, [off], idxv, mask=lastm & valid)
                    return c
                lax.fori_loop(0, (tt + WIN - 1) // WIN, wb_, 0)

            winner(rg_ref, stg, ttg, Wg)
            winner(rc_ref, stc, ttc, Wc)

            def ebody(w, c):
                wb = pl.multiple_of(stc + w * WIN, 8)
                pltpu.sync_copy(rc_ref.at[pl.ds(wb, WIN)], rwin)
                for g in range(WIN // L):
                    sl = pl.ds(g * L, L)
                    rv = rwin[sl]
                    off = rv & (BSZ - 1)
                    hj = plsc.load_gather(Wg, [off])
                    xi = plsc.load_gather(Wc, [off])
                    hfb[sl] = hj
                    hjb[sl] = jnp.clip(hj, 0, NG - 1)
                    xib[sl] = jnp.clip(xi, 0, NC - 1)
                d1 = pltpu.async_copy(gv_ref.at[hjb], hrow, sem)
                d2 = pltpu.async_copy(cv_ref.at[xib], xrow, sem)
                d1.wait()
                d2.wait()
                pltpu.sync_copy(hrow, h_ref.at[pl.ds(wb, WIN)])
                pltpu.sync_copy(xrow, x_ref.at[pl.ds(wb, WIN)])
                pltpu.sync_copy(hfb, hf_ref.at[pl.ds(wb, WIN)])
                return c
            lax.fori_loop(0, (ttc + WIN - 1) // WIN, ebody, 0)
            return cc
        lax.fori_loop(0, NBPT, bucket, 0)

    R5 = RC_SZ // 8
    B5 = 1024

    def gru_body(h_ref, x_ref, hj_ref, e_ref, wz_r, uz_r, wr_r, ur_r,
                 wn_r, un_r, bz_r, br_r, bn_r, o_ref):
        m = (hj_ref[...] >= 0).astype(jnp.float32)

        def mm(a, w):
            return lax.dot_general(a, w, (((1,), (0,)), ((), ())),
                                   preferred_element_type=jnp.float32,
                                   precision=lax.Precision.HIGHEST)
        m128 = mm(m, e_ref[...])
        h = h_ref[...] * m128
        x = x_ref[...]
        z = jax.nn.sigmoid(mm(x, wz_r[...]) + mm(h, uz_r[...]) + bz_r[...])
        r = jax.nn.sigmoid(mm(x, wr_r[...]) + mm(h, ur_r[...]) + br_r[...])
        nn = jnp.tanh(mm(x, wn_r[...]) + mm(r * h, un_r[...]) + bn_r[...])
        o_ref[...] = (1.0 - z) * h + z * nn

    full = lambda s: pl.BlockSpec(s, lambda i: (0, 0))
    k4 = pl.pallas_call(
        gru_body,
        grid=(pl.cdiv(R5, B5),),
        in_specs=[pl.BlockSpec((B5, 128), lambda i: (i, 0)),
                  pl.BlockSpec((B5, 128), lambda i: (i, 0)),
                  pl.BlockSpec((B5, 8), lambda i: (i, 0)),
                  full((8, 128)),
                  full((128, 128)), full((128, 128)), full((128, 128)),
                  full((128, 128)), full((128, 128)), full((128, 128)),
                  full((1, 128)), full((1, 128)), full((1, 128))],
        out_specs=pl.BlockSpec((B5, 128), lambda i: (i, 0)),
        out_shape=jax.ShapeDtypeStruct((R5, 128), jnp.float32),
    )

    @functools.partial(
        pl.kernel,
        out_type=jax.ShapeDtypeStruct((LINSZ, CCH), jnp.float32),
        mesh=mesh,
        compiler_params=pltpu.CompilerParams(needs_layout_passes=False, use_tc_tiling_on_sc=False),
        scratch_types=[pltpu.VMEM((nw, HP), jnp.int32),
                       pltpu.VMEM((HP,), jnp.int32),
                       pltpu.VMEM((HP,), jnp.int32),
                       pltpu.VMEM((HP,), jnp.int32),
                       pltpu.VMEM((HP,), jnp.int32),
                       pltpu.VMEM((BSZ,), jnp.int32),
                       pltpu.VMEM((BSZ,), jnp.int32),
                       pltpu.VMEM((WIN,), jnp.int32),
                       pltpu.VMEM((BSZ,), jnp.int32),
                       pltpu.VMEM((BSZ,), jnp.int32),
                       pltpu.VMEM((BSZ // WIN, WIN), jnp.int32),
                       pltpu.VMEM((BSZ // WIN, WIN), jnp.int32),
                       pltpu.VMEM((BSZ // WIN, WIN), jnp.int32),
                       pltpu.VMEM((BSZ, CCH), jnp.float32),
                       pltpu.VMEM((WIN, CCH), jnp.float32),
                       pltpu.SemaphoreType.DMA],
    )
    def k5(rg_ref, rc_ref, hg_ref, hc_ref, gv_ref, hn_ref, vol_ref,
           histall, stg_b, ttg_b, stc_b, ttc_b, Wg, Wc, rwin,
           gg, hgb, gc2d, hc2d, zi2d, grows, zrow, sem):
        wid = lax.axis_index("s") * info.num_cores + lax.axis_index("c")
        i16 = lax.iota(jnp.int32, L)
        _starts_tots(hg_ref, histall, stg_b, ttg_b, nw)
        _starts_tots(hc_ref, histall, stc_b, ttc_b, nw)

        def zb(q, c):
            for j in range(L):
                zrow[q * L + j, :] = jnp.zeros((L,), jnp.float32)
            return c
        lax.fori_loop(0, WIN // L, zb, 0)

        def bucket(it, cc):
            kb = wid + it * nw
            cell0 = kb * BSZ
            stg = _sload(stg_b, kb)
            ttg = _sload(ttg_b, kb)
            stc = _sload(stc_b, kb)
            ttc = _sload(ttc_b, kb)

            def ib(q, c):
                for j in range(L):
                    sl = pl.ds((q * L + j) * L, L)
                    Wg[sl] = jnp.full((L,), -1, jnp.int32)
                    Wc[sl] = jnp.full((L,), -1, jnp.int32)
                return c
            lax.fori_loop(0, BSZ // (L * L), ib, 0)

            def wgb(w, c):
                wb = pl.multiple_of(stg + w * WIN, 8)
                pltpu.sync_copy(rg_ref.at[pl.ds(wb, WIN)], rwin)
                rel = wb - stg
                for g in range(WIN // L):
                    rv = rwin[pl.ds(g * L, L)]
                    e = rel + g * L + i16
                    valid = e < ttg
                    off = rv & (BSZ - 1)
                    idxv = rv >> BSH
                    cnt, lastm = plsc.scan_count(off, mask=valid)
                    plsc.store_scatter(Wg, [off], idxv, mask=lastm & valid)
                return c
            lax.fori_loop(0, (ttg + WIN - 1) // WIN, wgb, 0)

            def wcb(w, c):
                wb = pl.multiple_of(stc + w * WIN, 8)
                pltpu.sync_copy(rc_ref.at[pl.ds(wb, WIN)], rwin)
                rel = wb - stc
                for g in range(WIN // L):
                    rv = rwin[pl.ds(g * L, L)]
                    e = rel + g * L + i16
                    valid = e < ttc
                    off = rv & (BSZ - 1)
                    pos = wb + g * L + i16
                    cnt, lastm = plsc.scan_count(off, mask=valid)
                    plsc.store_scatter(Wc, [off], pos, mask=lastm & valid)
                return c
            lax.fori_loop(0, (ttc + WIN - 1) // WIN, wcb, 0)

            def bp(q, c):
                for j8 in range(8):
                    v = q * 8 + j8
                    sl = pl.ds(v * L, L)
                    wg = Wg[sl]
                    wc = Wc[sl]
                    cella = cell0 + v * L + i16
                    hasg = wg >= 0
                    hasc = wc >= 0
                    inb = cella < LINSZ
                    gsel = hasg & (~hasc) & inb
                    hsel = hasc & inb
                    zsel = (~hasg) & (~hasc) & inb
                    gg[sl] = jnp.where(gsel, wg, -1)
                    hgb[sl] = jnp.where(hsel, wc, -1)
                    row = v // (WIN // L)
                    col = pl.ds((v % (WIN // L)) * L, L)
                    gc2d[row, col] = jnp.where(gsel, cella, -1)
                    hc2d[row, col] = jnp.where(hsel, cella, -1)
                    zi2d[row, col] = jnp.where(zsel, cella, -1)
                return c
            lax.fori_loop(0, BSZ // (L * 8), bp, 0)

            pltpu.async_copy(
                gv_ref.at[plsc.Indices(gg, ignored_value=-1)], grows,
                sem).wait()
            descs = []
            for c8 in range(BSZ // WIN):
                descs.append(pltpu.async_copy(
                    grows.at[pl.ds(c8 * WIN, WIN)],
                    vol_ref.at[plsc.Indices(gc2d.at[c8], ignored_value=-1)],
                    sem))
            for d in descs:
                d.wait()
            pltpu.async_copy(
                hn_ref.at[plsc.Indices(hgb, ignored_value=-1)], grows,
                sem).wait()
            descs = []
            for c8 in range(BSZ // WIN):
                descs.append(pltpu.async_copy(
                    grows.at[pl.ds(c8 * WIN, WIN)],
                    vol_ref.at[plsc.Indices(hc2d.at[c8], ignored_value=-1)],
                    sem))
                descs.append(pltpu.async_copy(
                    zrow,
                    vol_ref.at[plsc.Indices(zi2d.at[c8], ignored_value=-1)],
                    sem))
            for d in descs:
                d.wait()
            return cc
        lax.fori_loop(0, NBPT, bucket, 0)

    cg = global_coords.astype(jnp.int32).reshape(-1)
    cc = current_coords.astype(jnp.int32).reshape(-1)
    cg = jnp.pad(cg, (0, NGP * 3 - NG * 3))
    cc = jnp.pad(cc, (0, NCP * 3 - NC * 3))

    lin_g, lin_c, hist_g, hist_c = k1(cg, cc)
    routed_g, routed_c = k2(lin_g, lin_c, hist_g, hist_c)
    h, x, hjf = k3(routed_g, routed_c, hist_g, hist_c,
                   global_values, current_values)

    eye8 = jnp.eye(8, dtype=jnp.float32)
    kron = lambda w: jnp.kron(eye8, w)
    emask = jnp.kron(eye8, jnp.ones((1, CCH), jnp.float32))
    hn2 = k4(h.reshape(R5, 128), x.reshape(R5, 128), hjf.reshape(R5, 8),
             emask, kron(Wz), kron(Uz), kron(Wr), kron(Ur), kron(Wn),
             kron(Un), jnp.tile(bz, 8)[None], jnp.tile(br, 8)[None],
             jnp.tile(bn, 8)[None])
    hnew = hn2.reshape(RC_SZ, CCH)

    vol = k5(routed_g, routed_c, hist_g, hist_c, global_values, hnew)
    return vol.reshape(DIM, DIM, DIM, CCH)


# speedup vs baseline: 5.8140x; 1.5902x over previous
import functools

import jax
import jax.numpy as jnp
from jax import lax
from jax.experimental import pallas as pl
from jax.experimental.pallas import tpu as pltpu
from jax.experimental.pallas import tpu_sc as plsc

DIM = 96
D2 = DIM * DIM
LINSZ = DIM ** 3
CCH = 16
NBKT = 216
BSH = 12
BSZ = 1 << BSH
HP = 240
NG = 500000
NC = 200000
WIN = 1024
WINR = 2048
SCH = 512
L = 16


def _sload(ref, idx):
    return ref[pl.ds(idx, L)][0]


def _starts_tots(h_ref, histall, st_buf, tt_buf, nw):
    pltpu.sync_copy(h_ref, histall)
    carry = jnp.int32(0)
    for kb in range(HP // L):
        sl = pl.ds(kb * L, L)
        tot = jnp.zeros((L,), jnp.int32)
        for t in range(nw):
            tot = tot + histall[t, sl]
        ptot = jnp.bitwise_and(tot + (WIN - 1), -WIN)
        cum = plsc.cumsum(ptot)
        st_buf[sl] = cum - ptot + carry
        tt_buf[sl] = tot
        carry = carry + jnp.sum(ptot)


def _my_bases(wid, histall, st_buf, brun, nw):
    for kb in range(HP // L):
        sl = pl.ds(kb * L, L)
        myb = st_buf[sl]
        for t in range(nw):
            myb = myb + jnp.where(jnp.int32(t) < wid, histall[t, sl],
                                  jnp.zeros((L,), jnp.int32))
        brun[sl] = myb


def kernel(global_coords, global_values, current_coords, current_values,
           Wz, Uz, Wr, Ur, Wn, Un, bz, br, bn):
    info = plsc.get_sparse_core_info()
    nw = info.num_cores * info.num_subcores
    mesh = plsc.VectorSubcoreMesh(core_axis_name="c", subcore_axis_name="s")

    def roundup(a, b):
        return (a + b - 1) // b * b

    CH_G = roundup((NG + nw - 1) // nw, WINR)
    CH_C = roundup((NC + nw - 1) // nw, WINR)
    NGP = CH_G * nw
    NCP = CH_C * nw
    RG_SZ = NG + NBKT * WIN + WIN
    RC_SZ = NC + NBKT * WIN + WIN
    NBPT = (NBKT + nw - 1) // nw

    @functools.partial(
        pl.kernel,
        out_type=(jax.ShapeDtypeStruct((NGP,), jnp.int32),
                  jax.ShapeDtypeStruct((NCP,), jnp.int32),
                  jax.ShapeDtypeStruct((nw, HP), jnp.int32),
                  jax.ShapeDtypeStruct((nw, HP), jnp.int32)),
        mesh=mesh,
        compiler_params=pltpu.CompilerParams(needs_layout_passes=False, use_tc_tiling_on_sc=False),
        scratch_types=[pltpu.VMEM((WINR,), jnp.int32),
                       pltpu.VMEM((WINR,), jnp.int32),
                       pltpu.VMEM((WINR,), jnp.int32),
                       pltpu.VMEM((WINR,), jnp.int32),
                       pltpu.VMEM((HP,), jnp.int32),
                       pltpu.SemaphoreType.DMA],
    )
    def k1(cg_ref, cc_ref, lg_ref, lc_ref, hg_ref, hc_ref,
           xbuf, ybuf, zbuf, lbuf, hbuf, sem):
        wid = lax.axis_index("s") * info.num_cores + lax.axis_index("c")
        i16 = lax.iota(jnp.int32, L)

        def phase(c_ref, lin_ref, hrow_ref, n, chk):
            def zb(q, c):
                hbuf[pl.ds(q * L, L)] = jnp.zeros((L,), jnp.int32)
                return c
            lax.fori_loop(0, HP // L, zb, 0)
            base = wid * chk

            def wbody(w, c):
                wb = pl.multiple_of(base + w * WINR, WINR)
                d1 = pltpu.async_copy(c_ref.at[0, pl.ds(wb, WINR)], xbuf, sem)
                d2 = pltpu.async_copy(c_ref.at[1, pl.ds(wb, WINR)], ybuf, sem)
                d3 = pltpu.async_copy(c_ref.at[2, pl.ds(wb, WINR)], zbuf, sem)
                d1.wait(); d2.wait(); d3.wait()
                for g in range(WINR // L):
                    sl = pl.ds(g * L, L)
                    linv = xbuf[sl] * D2 + ybuf[sl] * DIM + zbuf[sl]
                    lbuf[sl] = linv
                    valid = (wb + g * L + i16) < n
                    key = jnp.minimum(linv >> BSH, NBKT - 1)
                    gth = plsc.load_gather(hbuf, [key], mask=valid)
                    cnt, lastm = plsc.scan_count(key, mask=valid)
                    plsc.store_scatter(hbuf, [key], gth + cnt, mask=lastm & valid)
                pltpu.sync_copy(lbuf, lin_ref.at[pl.ds(wb, WINR)])
                return c
            lax.fori_loop(0, chk // WINR, wbody, 0)
            pltpu.sync_copy(hbuf, hrow_ref)

        phase(cg_ref, lg_ref, hg_ref.at[wid], NG, CH_G)
        phase(cc_ref, lc_ref, hc_ref.at[wid], NC, CH_C)

    @functools.partial(
        pl.kernel,
        out_type=(jax.ShapeDtypeStruct((RG_SZ,), jnp.int32),
                  jax.ShapeDtypeStruct((RC_SZ,), jnp.int32)),
        mesh=mesh,
        compiler_params=pltpu.CompilerParams(needs_layout_passes=False, use_tc_tiling_on_sc=False),
        scratch_types=[pltpu.VMEM((nw, HP), jnp.int32),
                       pltpu.VMEM((HP,), jnp.int32),
                       pltpu.VMEM((HP,), jnp.int32),
                       pltpu.VMEM((HP,), jnp.int32),
                       pltpu.VMEM((WINR,), jnp.int32),
                       pltpu.VMEM((WINR // SCH, SCH), jnp.int32),
                       pltpu.VMEM((WINR,), jnp.int32),
                       pltpu.SemaphoreType.DMA],
    )
    def k2(lg_ref, lc_ref, hg_ref, hc_ref, rg_ref, rc_ref,
           histall, st_buf, tt_buf, brun, lwin, posb, valb, sem):
        wid = lax.axis_index("s") * info.num_cores + lax.axis_index("c")
        i16 = lax.iota(jnp.int32, L)

        def phase(h_ref, lin_ref, routed_ref, n, chk):
            _starts_tots(h_ref, histall, st_buf, tt_buf, nw)
            _my_bases(wid, histall, st_buf, brun, nw)

            nsub = SCH // L

            def wbody(w, c):
                wb = pl.multiple_of(wid * chk + w * WINR, WINR)
                pltpu.sync_copy(lin_ref.at[pl.ds(wb, WINR)], lwin)
                for g in range(WINR // L):
                    linv = lwin[pl.ds(g * L, L)]
                    j = wb + g * L + i16
                    valid = j < n
                    key = jnp.minimum(linv >> BSH, NBKT - 1)
                    off = linv & (BSZ - 1)
                    gb = plsc.load_gather(brun, [key], mask=valid)
                    cnt, lastm = plsc.scan_count(key, mask=valid)
                    plsc.store_scatter(brun, [key], gb + cnt, mask=lastm & valid)
                    posb[g // nsub, pl.ds((g % nsub) * L, L)] = (
                        jnp.where(valid, gb + cnt - 1, -1))
                    valb[pl.ds(g * L, L)] = j * BSZ + off
                descs = []
                for c8 in range(WINR // SCH):
                    descs.append(pltpu.async_copy(
                        valb.at[pl.ds(c8 * SCH, SCH)],
                        routed_ref.at[plsc.Indices(posb.at[c8],
                                                   ignored_value=-1)],
                        sem))
                for d in descs:
                    d.wait()
                return c
            lax.fori_loop(0, chk // WINR, wbody, 0)

        phase(hg_ref, lg_ref, rg_ref, NG, CH_G)
        phase(hc_ref, lc_ref, rc_ref, NC, CH_C)

    @functools.partial(
        pl.kernel,
        out_type=(jax.ShapeDtypeStruct((RC_SZ, CCH), jnp.float32),
                  jax.ShapeDtypeStruct((RC_SZ, CCH), jnp.float32),
                  jax.ShapeDtypeStruct((RC_SZ,), jnp.int32)),
        mesh=mesh,
        compiler_params=pltpu.CompilerParams(needs_layout_passes=False, use_tc_tiling_on_sc=False),
        scratch_types=[pltpu.VMEM((nw, HP), jnp.int32),
                       pltpu.VMEM((HP,), jnp.int32),
                       pltpu.VMEM((HP,), jnp.int32),
                       pltpu.VMEM((HP,), jnp.int32),
                       pltpu.VMEM((HP,), jnp.int32),
                       pltpu.VMEM((BSZ,), jnp.int32),
                       pltpu.VMEM((BSZ,), jnp.int32),
                       pltpu.VMEM((WIN,), jnp.int32),
                       pltpu.VMEM((WIN,), jnp.int32),
                       pltpu.VMEM((WIN,), jnp.int32),
                       pltpu.VMEM((WIN,), jnp.int32),
                       pltpu.VMEM((WIN, CCH), jnp.float32),
                       pltpu.VMEM((WIN, CCH), jnp.float32),
                       pltpu.SemaphoreType.DMA],
    )
    def k3(rg_ref, rc_ref, hg_ref, hc_ref, gv_ref, cv_ref,
           h_ref, x_ref, hf_ref,
           histall, stg_b, ttg_b, stc_b, ttc_b, Wg, Wc, rwin,
           hfb, hjb, xib, hrow, xrow, sem):
        wid = lax.axis_index("s") * info.num_cores + lax.axis_index("c")
        i16 = lax.iota(jnp.int32, L)
        _starts_tots(hg_ref, histall, stg_b, ttg_b, nw)
        _starts_tots(hc_ref, histall, stc_b, ttc_b, nw)

        def bucket(it, cc):
            kb = wid + it * nw
            stg = _sload(stg_b, kb)
            ttg = _sload(ttg_b, kb)
            stc = _sload(stc_b, kb)
            ttc = _sload(ttc_b, kb)

            def ib(q, c):
                for j in range(L):
                    sl = pl.ds((q * L + j) * L, L)
                    Wg[sl] = jnp.full((L,), -1, jnp.int32)
                    Wc[sl] = jnp.full((L,), -1, jnp.int32)
                return c
            lax.fori_loop(0, BSZ // (L * L), ib, 0)

            def winner(routed_ref, st, tt, wref):
                def wb_(w, c):
                    wb = pl.multiple_of(st + w * WIN, 8)
                    pltpu.sync_copy(routed_ref.at[pl.ds(wb, WIN)], rwin)
                    rel = wb - st
                    for g in range(WIN // L):
                        rv = rwin[pl.ds(g * L, L)]
                        e = rel + g * L + i16
                        valid = e < tt
                        off = rv & (BSZ - 1)
                        idxv = rv >> BSH
                        cnt, lastm = plsc.scan_count(off, mask=valid)
                        plsc.store_scatter(wref, [off], idxv, mask=lastm & valid)
                    return c
                lax.fori_loop(0, (tt + WIN - 1) // WIN, wb_, 0)

            winner(rg_ref, stg, ttg, Wg)
            winner(rc_ref, stc, ttc, Wc)

            def ebody(w, c):
                wb = pl.multiple_of(stc + w * WIN, 8)
                pltpu.sync_copy(rc_ref.at[pl.ds(wb, WIN)], rwin)
                for g in range(WIN // L):
                    sl = pl.ds(g * L, L)
                    rv = rwin[sl]
                    off = rv & (BSZ - 1)
                    hj = plsc.load_gather(Wg, [off])
                    xi = plsc.load_gather(Wc, [off])
                    hfb[sl] = hj
                    hjb[sl] = jnp.clip(hj, 0, NG - 1)
                    xib[sl] = jnp.clip(xi, 0, NC - 1)
                d1 = pltpu.async_copy(gv_ref.at[hjb], hrow, sem)
                d2 = pltpu.async_copy(cv_ref.at[xib], xrow, sem)
                d1.wait()
                d2.wait()
                pltpu.sync_copy(hrow, h_ref.at[pl.ds(wb, WIN)])
                pltpu.sync_copy(xrow, x_ref.at[pl.ds(wb, WIN)])
                pltpu.sync_copy(hfb, hf_ref.at[pl.ds(wb, WIN)])
                return c
            lax.fori_loop(0, (ttc + WIN - 1) // WIN, ebody, 0)
            return cc
        lax.fori_loop(0, NBPT, bucket, 0)

    R5 = RC_SZ // 8
    B5 = 1024

    def gru_body(h_ref, x_ref, hj_ref, e_ref, wz_r, uz_r, wr_r, ur_r,
                 wn_r, un_r, bz_r, br_r, bn_r, o_ref):
        m = (hj_ref[...] >= 0).astype(jnp.float32)

        def mm(a, w):
            return lax.dot_general(a, w, (((1,), (0,)), ((), ())),
                                   preferred_element_type=jnp.float32,
                                   precision=lax.Precision.HIGHEST)
        m128 = mm(m, e_ref[...])
        h = h_ref[...] * m128
        x = x_ref[...]
        z = jax.nn.sigmoid(mm(x, wz_r[...]) + mm(h, uz_r[...]) + bz_r[...])
        r = jax.nn.sigmoid(mm(x, wr_r[...]) + mm(h, ur_r[...]) + br_r[...])
        nn = jnp.tanh(mm(x, wn_r[...]) + mm(r * h, un_r[...]) + bn_r[...])
        o_ref[...] = (1.0 - z) * h + z * nn

    full = lambda s: pl.BlockSpec(s, lambda i: (0, 0))
    k4 = pl.pallas_call(
        gru_body,
        grid=(pl.cdiv(R5, B5),),
        in_specs=[pl.BlockSpec((B5, 128), lambda i: (i, 0)),
                  pl.BlockSpec((B5, 128), lambda i: (i, 0)),
                  pl.BlockSpec((B5, 8), lambda i: (i, 0)),
                  full((8, 128)),
                  full((128, 128)), full((128, 128)), full((128, 128)),
                  full((128, 128)), full((128, 128)), full((128, 128)),
                  full((1, 128)), full((1, 128)), full((1, 128))],
        out_specs=pl.BlockSpec((B5, 128), lambda i: (i, 0)),
        out_shape=jax.ShapeDtypeStruct((R5, 128), jnp.float32),
    )

    @functools.partial(
        pl.kernel,
        out_type=jax.ShapeDtypeStruct((LINSZ, CCH), jnp.float32),
        mesh=mesh,
        compiler_params=pltpu.CompilerParams(needs_layout_passes=False, use_tc_tiling_on_sc=False),
        scratch_types=[pltpu.VMEM((nw, HP), jnp.int32),
                       pltpu.VMEM((HP,), jnp.int32),
                       pltpu.VMEM((HP,), jnp.int32),
                       pltpu.VMEM((HP,), jnp.int32),
                       pltpu.VMEM((HP,), jnp.int32),
                       pltpu.VMEM((BSZ,), jnp.int32),
                       pltpu.VMEM((BSZ,), jnp.int32),
                       pltpu.VMEM((WIN,), jnp.int32),
                       pltpu.VMEM((BSZ,), jnp.int32),
                       pltpu.VMEM((BSZ,), jnp.int32),
                       pltpu.VMEM((BSZ // SCH, SCH), jnp.int32),
                       pltpu.VMEM((BSZ // SCH, SCH), jnp.int32),
                       pltpu.VMEM((BSZ // SCH, SCH), jnp.int32),
                       pltpu.VMEM((BSZ, CCH), jnp.float32),
                       pltpu.VMEM((SCH, CCH), jnp.float32),
                       pltpu.SemaphoreType.DMA],
    )
    def k5(rg_ref, rc_ref, hg_ref, hc_ref, gv_ref, hn_ref, vol_ref,
           histall, stg_b, ttg_b, stc_b, ttc_b, Wg, Wc, rwin,
           gg, hgb, gc2d, hc2d, zi2d, grows, zrow, sem):
        wid = lax.axis_index("s") * info.num_cores + lax.axis_index("c")
        i16 = lax.iota(jnp.int32, L)
        _starts_tots(hg_ref, histall, stg_b, ttg_b, nw)
        _starts_tots(hc_ref, histall, stc_b, ttc_b, nw)

        def zb(q, c):
            for j in range(L):
                zrow[q * L + j, :] = jnp.zeros((L,), jnp.float32)
            return c
        lax.fori_loop(0, SCH // L, zb, 0)

        def bucket(it, cc):
            kb = wid + it * nw
            cell0 = kb * BSZ
            stg = _sload(stg_b, kb)
            ttg = _sload(ttg_b, kb)
            stc = _sload(stc_b, kb)
            ttc = _sload(ttc_b, kb)

            def ib(q, c):
                for j in range(L):
                    sl = pl.ds((q * L + j) * L, L)
                    Wg[sl] = jnp.full((L,), -1, jnp.int32)
                    Wc[sl] = jnp.full((L,), -1, jnp.int32)
                return c
            lax.fori_loop(0, BSZ // (L * L), ib, 0)

            def wgb(w, c):
                wb = pl.multiple_of(stg + w * WIN, 8)
                pltpu.sync_copy(rg_ref.at[pl.ds(wb, WIN)], rwin)
                rel = wb - stg
                for g in range(WIN // L):
                    rv = rwin[pl.ds(g * L, L)]
                    e = rel + g * L + i16
                    valid = e < ttg
                    off = rv & (BSZ - 1)
                    idxv = rv >> BSH
                    cnt, lastm = plsc.scan_count(off, mask=valid)
                    plsc.store_scatter(Wg, [off], idxv, mask=lastm & valid)
                return c
            lax.fori_loop(0, (ttg + WIN - 1) // WIN, wgb, 0)

            def wcb(w, c):
                wb = pl.multiple_of(stc + w * WIN, 8)
                pltpu.sync_copy(rc_ref.at[pl.ds(wb, WIN)], rwin)
                rel = wb - stc
                for g in range(WIN // L):
                    rv = rwin[pl.ds(g * L, L)]
                    e = rel + g * L + i16
                    valid = e < ttc
                    off = rv & (BSZ - 1)
                    pos = wb + g * L + i16
                    cnt, lastm = plsc.scan_count(off, mask=valid)
                    plsc.store_scatter(Wc, [off], pos, mask=lastm & valid)
                return c
            lax.fori_loop(0, (ttc + WIN - 1) // WIN, wcb, 0)

            def bp(q, c):
                for j8 in range(8):
                    v = q * 8 + j8
                    sl = pl.ds(v * L, L)
                    wg = Wg[sl]
                    wc = Wc[sl]
                    cella = cell0 + v * L + i16
                    hasg = wg >= 0
                    hasc = wc >= 0
                    inb = cella < LINSZ
                    gsel = hasg & (~hasc) & inb
                    hsel = hasc & inb
                    zsel = (~hasg) & (~hasc) & inb
                    gg[sl] = jnp.where(gsel, wg, -1)
                    hgb[sl] = jnp.where(hsel, wc, -1)
                    row = v // (SCH // L)
                    col = pl.ds((v % (SCH // L)) * L, L)
                    gc2d[row, col] = jnp.where(gsel, cella, -1)
                    hc2d[row, col] = jnp.where(hsel, cella, -1)
                    zi2d[row, col] = jnp.where(zsel, cella, -1)
                return c
            lax.fori_loop(0, BSZ // (L * 8), bp, 0)

            pltpu.async_copy(
                gv_ref.at[plsc.Indices(gg, ignored_value=-1)], grows,
                sem).wait()
            descs = []
            for c8 in range(BSZ // SCH):
                descs.append(pltpu.async_copy(
                    grows.at[pl.ds(c8 * SCH, SCH)],
                    vol_ref.at[plsc.Indices(gc2d.at[c8], ignored_value=-1)],
                    sem))
            for d in descs:
                d.wait()
            pltpu.async_copy(
                hn_ref.at[plsc.Indices(hgb, ignored_value=-1)], grows,
                sem).wait()
            descs = []
            for c8 in range(BSZ // SCH):
                descs.append(pltpu.async_copy(
                    grows.at[pl.ds(c8 * SCH, SCH)],
                    vol_ref.at[plsc.Indices(hc2d.at[c8], ignored_value=-1)],
                    sem))
                descs.append(pltpu.async_copy(
                    zrow,
                    vol_ref.at[plsc.Indices(zi2d.at[c8], ignored_value=-1)],
                    sem))
            for d in descs:
                d.wait()
            return cc
        lax.fori_loop(0, NBPT, bucket, 0)

    cg = jnp.pad(global_coords.astype(jnp.int32).T, ((0, 0), (0, NGP - NG)))
    cc = jnp.pad(current_coords.astype(jnp.int32).T, ((0, 0), (0, NCP - NC)))

    lin_g, lin_c, hist_g, hist_c = k1(cg, cc)
    routed_g, routed_c = k2(lin_g, lin_c, hist_g, hist_c)
    h, x, hjf = k3(routed_g, routed_c, hist_g, hist_c,
                   global_values, current_values)

    eye8 = jnp.eye(8, dtype=jnp.float32)
    kron = lambda w: jnp.kron(eye8, w)
    emask = jnp.kron(eye8, jnp.ones((1, CCH), jnp.float32))
    hn2 = k4(h.reshape(R5, 128), x.reshape(R5, 128), hjf.reshape(R5, 8),
             emask, kron(Wz), kron(Uz), kron(Wr), kron(Ur), kron(Wn),
             kron(Un), jnp.tile(bz, 8)[None], jnp.tile(br, 8)[None],
             jnp.tile(bn, 8)[None])
    hnew = hn2.reshape(RC_SZ, CCH)

    vol = k5(routed_g, routed_c, hist_g, hist_c, global_values, hnew)
    return vol.reshape(DIM, DIM, DIM, CCH)
